# Initial kernel scaffold; baseline (speedup 1.0000x reference)
import jax
import jax.numpy as jnp
from jax.experimental import pallas as pl


def kernel(x, edge_index, edge_attr, batch, kern1, W1a, b1a, W2a, b2a, kern2, W1b, b1b, W2b, b2b, conv_w, conv_b, fc_w, fc_b):
    raise NotImplementedError("write your pallas kernel here")


# speedup vs baseline: 12.6507x; 12.6507x over previous
import jax
import jax.numpy as jnp
from jax import lax
from jax.experimental import pallas as pl
from jax.experimental.pallas import tpu as pltpu

N_GRAPHS = 4096
N_NODES = 32
WINDOW = 128
N_EDGES = 194
GRAPHS_PER_BLOCK = 32
ROWS_PER_BLOCK = GRAPHS_PER_BLOCK * N_NODES
GRID = N_GRAPHS // GRAPHS_PER_BLOCK


def _fold_conv(W, kref, K):
    pad_l = (K - 1) // 2
    C, H = W.shape
    acc = W * kref[0, pad_l]
    for k in range(K):
        s = k - pad_l
        if s == 0:
            continue
        if s > 0:
            sh = jnp.concatenate([jnp.zeros((s, H), W.dtype), W[: C - s, :]], axis=0)
        else:
            sh = jnp.concatenate([W[-s:, :], jnp.zeros((-s, H), W.dtype)], axis=0)
        acc = acc + kref[0, k] * sh
    return acc


def _blockdiag2(W):
    K, H = W.shape
    z = jnp.zeros((K, H), W.dtype)
    top = jnp.concatenate([W, z], axis=1)
    bot = jnp.concatenate([z, W], axis=1)
    return jnp.concatenate([top, bot], axis=0)


def _tc_body(x_ref, eiT_ref, ew_ref, k1_ref, W1a_ref, b1a_ref, W2a_ref,
             b2a_ref, k2_ref, W1b_ref, b1b_ref, W2b_ref, b2b_ref, cw_ref,
             cb_ref, fcw_ref, fcb_ref, out_ref):
    f32 = jnp.float32

    e_iota = lax.broadcasted_iota(jnp.int32, (256, N_NODES), 0)
    n_iota = lax.broadcasted_iota(jnp.int32, (256, N_NODES), 1)
    src = eiT_ref[:, 0:1]
    dst = eiT_ref[:, 1:2]
    valid = e_iota < N_EDGES
    D = jnp.where((n_iota == dst) & valid, 1.0, 0.0).astype(f32)
    S = jnp.where((n_iota == src) & valid, 1.0, 0.0).astype(f32) * ew_ref[:, :]
    A = lax.dot_general(D, S, (((0,), (0,)), ((), ())),
                        preferred_element_type=f32)
    r32 = lax.broadcasted_iota(jnp.int32, (N_NODES, N_NODES), 0)
    c32 = lax.broadcasted_iota(jnp.int32, (N_NODES, N_NODES), 1)
    A = A + jnp.where(r32 == c32, 1.0, 0.0).astype(f32)

    T = jnp.concatenate([A, A, A, A], axis=0)
    T = jnp.concatenate([T, T, T, T], axis=1)
    rb = lax.broadcasted_iota(jnp.int32, (128, 128), 0) // N_NODES
    cb = lax.broadcasted_iota(jnp.int32, (128, 128), 1) // N_NODES
    A4 = jnp.where(rb == cb, T, 0.0).astype(f32)

    def mix(v):
        outs = []
        for c in range(ROWS_PER_BLOCK // 128):
            blk = v[c * 128:(c + 1) * 128, :]
            outs.append(lax.dot_general(A4, blk, (((1,), (0,)), ((), ())),
                                        preferred_element_type=f32))
        return jnp.concatenate(outs, axis=0)

    W1cat = _blockdiag2(_fold_conv(W1a_ref[...], k1_ref, 15))
    b1cat = jnp.concatenate([b1a_ref[...], b1a_ref[...]], axis=1)
    W2cat = _blockdiag2(W2a_ref[...])
    b2cat = jnp.concatenate([b2a_ref[...], b2a_ref[...]], axis=1)
    W3cat = _blockdiag2(_fold_conv(W1b_ref[...], k2_ref, 16))
    b3cat = jnp.concatenate([b1b_ref[...], b1b_ref[...]], axis=1)
    W4cat = _blockdiag2(W2b_ref[...])
    b4cat = jnp.concatenate([b2b_ref[...], b2b_ref[...]], axis=1)

    xb = x_ref[...]
    a1 = jax.nn.relu(jnp.dot(xb, W1cat, preferred_element_type=f32) + b1cat)
    h1 = jax.nn.relu(jnp.dot(mix(a1), W2cat, preferred_element_type=f32) + b2cat)
    a2 = jax.nn.relu(jnp.dot(h1, W3cat, preferred_element_type=f32) + b3cat)
    h2 = jax.nn.relu(jnp.dot(mix(a2), W4cat, preferred_element_type=f32) + b4cat)

    y = (jnp.dot(h2[:, :64], cw_ref[:, 0:1], preferred_element_type=f32) +
         jnp.dot(h2[:, 64:], cw_ref[:, 1:2], preferred_element_type=f32))
    y = jax.nn.relu(y + cb_ref[0, 0])

    Y = y.reshape(GRAPHS_PER_BLOCK, N_NODES)
    out = jax.nn.sigmoid(jnp.dot(Y, fcw_ref[...], preferred_element_type=f32)
                         + fcb_ref[0, 0])
    out_ref[...] = out


def kernel(x, edge_index, edge_attr, batch, kern1, W1a, b1a, W2a, b2a,
           kern2, W1b, b1b, W2b, b2b, conv_w, conv_b, fc_w, fc_b):
    del batch
    f32 = jnp.float32
    eiT = edge_index.T.astype(jnp.int32)
    ewc = edge_attr.reshape(256, 1).astype(f32)
    k1 = kern1.reshape(1, 15).astype(f32)
    k2 = kern2.reshape(1, 16).astype(f32)
    cw = conv_w[0].astype(f32)
    cb = conv_b.reshape(1, 1).astype(f32)
    fcb = fc_b.reshape(1, 1).astype(f32)

    vspec = lambda shape: pl.BlockSpec(shape, lambda i: (0, 0))
    sspec = lambda shape: pl.BlockSpec(shape, lambda i: (0, 0),
                                       memory_space=pltpu.SMEM)

    return pl.pallas_call(
        _tc_body,
        grid=(GRID,),
        in_specs=[
            pl.BlockSpec((ROWS_PER_BLOCK, 2 * WINDOW), lambda i: (i, 0)),
            vspec((256, 2)),
            vspec((256, 1)),
            sspec((1, 15)),
            vspec((WINDOW, 16)),
            vspec((1, 16)),
            vspec((16, 64)),
            vspec((1, 64)),
            sspec((1, 16)),
            vspec((64, 8)),
            vspec((1, 8)),
            vspec((8, 64)),
            vspec((1, 64)),
            vspec((64, 2)),
            sspec((1, 1)),
            vspec((N_NODES, 1)),
            sspec((1, 1)),
        ],
        out_specs=pl.BlockSpec((GRAPHS_PER_BLOCK, 1), lambda i: (i, 0)),
        out_shape=jax.ShapeDtypeStruct((N_GRAPHS, 1), f32),
        compiler_params=pltpu.CompilerParams(
            dimension_semantics=("parallel",)),
    )(x, eiT, ewc, k1, W1a, b1a.reshape(1, 16), W2a, b2a.reshape(1, 64),
      k2, W1b, b1b.reshape(1, 8), W2b, b2b.reshape(1, 64), cw, cb, fc_w, fcb)
